# initial kernel scaffold (unmeasured)
import functools

import jax
import jax.numpy as jnp
from jax import lax
from jax.experimental import pallas as pl
from jax.experimental.pallas import tpu as pltpu

N_DEV = 16


def kernel(x, router_W, route_idx, expert_W):
    n_tok, d_model = x.shape
    _, n_exp = router_W.shape
    e_per, _, d_out = expert_W.shape

    def body(x_ref, rw_ref, idx_ref, ew_ref, out_ref, wbuf, send_sems, recv_sems):
        me = lax.axis_index("i")

        barrier_sem = pltpu.get_barrier_semaphore()
        for k in range(1, N_DEV):
            pl.semaphore_signal(
                barrier_sem, inc=1,
                device_id=(lax.rem(me + k, N_DEV),),
                device_id_type=pl.DeviceIdType.MESH,
            )
        pl.semaphore_wait(barrier_sem, N_DEV - 1)

        rdmas = []
        for k in range(1, N_DEV):
            rdma = pltpu.make_async_remote_copy(
                src_ref=ew_ref,
                dst_ref=wbuf.at[k - 1],
                send_sem=send_sems.at[k - 1],
                recv_sem=recv_sems.at[k - 1],
                device_id=(lax.rem(me + k, N_DEV),),
                device_id_type=pl.DeviceIdType.MESH,
            )
            rdma.start()
            rdmas.append(rdma)

        scores = jnp.dot(x_ref[:, :], rw_ref[:, :],
                         preferred_element_type=jnp.float32)
        s_max = jnp.max(scores, axis=-1, keepdims=True)
        probs = jnp.exp(scores - s_max)
        probs = probs / jnp.sum(probs, axis=-1, keepdims=True)
        r0 = idx_ref[:, 0:1]
        r1 = idx_ref[:, 1:2]
        eids = lax.broadcasted_iota(jnp.int32, (1, n_exp), 1)
        g0 = jnp.sum(jnp.where(r0 == eids, probs, 0.0), axis=-1, keepdims=True)
        g1 = jnp.sum(jnp.where(r1 == eids, probs, 0.0), axis=-1, keepdims=True)
        gs = g0 + g1
        w0 = g0 / gs
        w1 = g1 / gs

        xv = x_ref[:, :]

        def contrib(origin, w_slab):
            acc = jnp.zeros((n_tok, d_out), jnp.float32)
            for j in range(e_per):
                e = origin * e_per + j
                coef = jnp.where(r0 == e, w0, 0.0) + jnp.where(r1 == e, w1, 0.0)
                acc = acc + jnp.dot(xv * coef, w_slab[j, :, :],
                                    preferred_element_type=jnp.float32)
            return acc

        out_ref[:, :] = contrib(me, ew_ref)

        for k in range(1, N_DEV):
            recv = pltpu.make_async_remote_copy(
                src_ref=ew_ref,
                dst_ref=wbuf.at[k - 1],
                send_sem=send_sems.at[k - 1],
                recv_sem=recv_sems.at[k - 1],
                device_id=(me,),
                device_id_type=pl.DeviceIdType.MESH,
            )
            recv.wait_recv()
            origin = lax.rem(me + (N_DEV - k), N_DEV)
            out_ref[:, :] += contrib(origin, wbuf.at[k - 1])

        for rdma in rdmas:
            rdma.wait_send()

        @functools.partial(pl.run_scoped, sem=pltpu.SemaphoreType.REGULAR)
        def _(sem):
            for k in range(1, N_DEV):
                pl.semaphore_signal(
                    sem, inc=1,
                    device_id=(lax.rem(me + k, N_DEV),),
                    device_id_type=pl.DeviceIdType.MESH,
                )
            pl.semaphore_wait(sem, N_DEV - 1)

    return pl.pallas_call(
        body,
        out_shape=jax.ShapeDtypeStruct((n_tok, d_out), jnp.float32),
        in_specs=[
            pl.BlockSpec(memory_space=pltpu.VMEM),
            pl.BlockSpec(memory_space=pltpu.VMEM),
            pl.BlockSpec(memory_space=pltpu.VMEM),
            pl.BlockSpec(memory_space=pltpu.VMEM),
        ],
        out_specs=pl.BlockSpec(memory_space=pltpu.VMEM),
        scratch_shapes=[
            pltpu.VMEM((N_DEV - 1, e_per, d_model, d_out), jnp.float32),
            pltpu.SemaphoreType.DMA((N_DEV - 1,)),
            pltpu.SemaphoreType.DMA((N_DEV - 1,)),
        ],
        compiler_params=pltpu.CompilerParams(collective_id=0),
    )(x, router_W, route_idx, expert_W)


# baseline (device time: 485849 ns/iter reference)
import functools

import jax
import jax.numpy as jnp
from jax import lax
from jax.experimental import pallas as pl
from jax.experimental.pallas import tpu as pltpu

N_DEV = 16


def kernel(x, router_W, route_idx, expert_W):
    n_tok, d_model = x.shape
    _, n_exp = router_W.shape
    e_per, _, d_out = expert_W.shape

    def body(x_ref, rw_ref, idx_ref, ew_ref, out_ref, wbuf, send_sems, recv_sems):
        me = lax.axis_index("i")

        barrier_sem = pltpu.get_barrier_semaphore()
        for k in range(1, N_DEV):
            pl.semaphore_signal(
                barrier_sem, inc=1,
                device_id=(lax.rem(me + k, N_DEV),),
                device_id_type=pl.DeviceIdType.MESH,
            )
        pl.semaphore_wait(barrier_sem, N_DEV - 1)

        rdmas = []
        for k in range(1, N_DEV):
            rdma = pltpu.make_async_remote_copy(
                src_ref=ew_ref,
                dst_ref=wbuf.at[k - 1],
                send_sem=send_sems.at[k - 1],
                recv_sem=recv_sems.at[k - 1],
                device_id=(lax.rem(me + k, N_DEV),),
                device_id_type=pl.DeviceIdType.MESH,
            )
            rdma.start()
            rdmas.append(rdma)

        scores = jnp.dot(x_ref[:, :], rw_ref[:, :],
                         preferred_element_type=jnp.float32)
        s_max = jnp.max(scores, axis=-1, keepdims=True)
        probs = jnp.exp(scores - s_max)
        probs = probs / jnp.sum(probs, axis=-1, keepdims=True)
        r0 = idx_ref[:, 0:1]
        r1 = idx_ref[:, 1:2]
        eids = lax.broadcasted_iota(jnp.int32, (1, n_exp), 1)
        g0 = jnp.sum(jnp.where(r0 == eids, probs, 0.0), axis=-1, keepdims=True)
        g1 = jnp.sum(jnp.where(r1 == eids, probs, 0.0), axis=-1, keepdims=True)
        gs = g0 + g1
        w0 = g0 / gs
        w1 = g1 / gs

        xv = x_ref[:, :]

        def contrib(origin, w_slab):
            acc = jnp.zeros((n_tok, d_out), jnp.float32)
            for j in range(e_per):
                e = origin * e_per + j
                coef = jnp.where(r0 == e, w0, 0.0) + jnp.where(r1 == e, w1, 0.0)
                acc = acc + jnp.dot(xv * coef, w_slab[j, :, :],
                                    preferred_element_type=jnp.float32)
            return acc

        out_ref[:, :] = contrib(me, ew_ref)

        for k in range(1, N_DEV):
            recv = pltpu.make_async_remote_copy(
                src_ref=ew_ref,
                dst_ref=wbuf.at[k - 1],
                send_sem=send_sems.at[k - 1],
                recv_sem=recv_sems.at[k - 1],
                device_id=(me,),
                device_id_type=pl.DeviceIdType.MESH,
            )
            recv.wait_recv()
            origin = lax.rem(me + (N_DEV - k), N_DEV)
            out_ref[:, :] += contrib(origin, wbuf.at[k - 1])

        for rdma in rdmas:
            rdma.wait_send()

        @functools.partial(pl.run_scoped, sem=pltpu.SemaphoreType.REGULAR)
        def _(sem):
            for k in range(1, N_DEV):
                pl.semaphore_signal(
                    sem, inc=1,
                    device_id=(lax.rem(me + k, N_DEV),),
                    device_id_type=pl.DeviceIdType.MESH,
                )
            pl.semaphore_wait(sem, N_DEV - 1)

    return pl.pallas_call(
        body,
        out_shape=jax.ShapeDtypeStruct((n_tok, d_out), jnp.float32),
        in_specs=[
            pl.BlockSpec(memory_space=pltpu.VMEM),
            pl.BlockSpec(memory_space=pltpu.VMEM),
            pl.BlockSpec(memory_space=pltpu.VMEM),
            pl.BlockSpec(memory_space=pltpu.VMEM),
        ],
        out_specs=pl.BlockSpec(memory_space=pltpu.VMEM),
        scratch_shapes=[
            pltpu.VMEM((N_DEV - 1, e_per, d_model, d_out), jnp.float32),
            pltpu.SemaphoreType.DMA((N_DEV - 1,)),
            pltpu.SemaphoreType.DMA((N_DEV - 1,)),
        ],
        compiler_params=pltpu.CompilerParams(
            collective_id=0,
            vmem_limit_bytes=64 * 1024 * 1024,
        ),
    )(x, router_W, route_idx, expert_W)


# device time: 243004 ns/iter; 1.9993x vs baseline; 1.9993x over previous
import functools

import jax
import jax.numpy as jnp
from jax import lax
from jax.experimental import pallas as pl
from jax.experimental.pallas import tpu as pltpu

N_DEV = 16


def kernel(x, router_W, route_idx, expert_W):
    n_tok, d_model = x.shape
    _, n_exp = router_W.shape
    e_per, _, d_out = expert_W.shape

    def body(x_ref, rw_ref, idx_ref, ew_ref, out_ref, wbuf, my_bf,
             send_sems, recv_sems):
        me = lax.axis_index("i")

        my_bf[:, :, :] = ew_ref[:, :, :].astype(jnp.bfloat16)

        barrier_sem = pltpu.get_barrier_semaphore()
        for k in range(1, N_DEV):
            pl.semaphore_signal(
                barrier_sem, inc=1,
                device_id=(lax.rem(me + k, N_DEV),),
                device_id_type=pl.DeviceIdType.MESH,
            )
        pl.semaphore_wait(barrier_sem, N_DEV - 1)

        rdmas = []
        for k in range(1, N_DEV):
            rdma = pltpu.make_async_remote_copy(
                src_ref=my_bf,
                dst_ref=wbuf.at[k - 1],
                send_sem=send_sems.at[k - 1],
                recv_sem=recv_sems.at[k - 1],
                device_id=(lax.rem(me + k, N_DEV),),
                device_id_type=pl.DeviceIdType.MESH,
            )
            rdma.start()
            rdmas.append(rdma)

        scores = jnp.dot(x_ref[:, :], rw_ref[:, :],
                         preferred_element_type=jnp.float32)
        s_max = jnp.max(scores, axis=-1, keepdims=True)
        probs = jnp.exp(scores - s_max)
        probs = probs / jnp.sum(probs, axis=-1, keepdims=True)
        r0 = idx_ref[:, 0:1]
        r1 = idx_ref[:, 1:2]
        eids = lax.broadcasted_iota(jnp.int32, (1, n_exp), 1)
        g0 = jnp.sum(jnp.where(r0 == eids, probs, 0.0), axis=-1, keepdims=True)
        g1 = jnp.sum(jnp.where(r1 == eids, probs, 0.0), axis=-1, keepdims=True)
        gs = g0 + g1
        w0 = g0 / gs
        w1 = g1 / gs

        xv = x_ref[:, :]

        def contrib(origin, w_slab):
            acc = jnp.zeros((n_tok, d_out), jnp.float32)
            for j in range(e_per):
                e = origin * e_per + j
                coef = jnp.where(r0 == e, w0, 0.0) + jnp.where(r1 == e, w1, 0.0)
                acc = acc + jnp.dot((xv * coef).astype(jnp.bfloat16),
                                    w_slab[j, :, :],
                                    preferred_element_type=jnp.float32)
            return acc

        out_ref[:, :] = contrib(me, my_bf)

        for k in range(1, N_DEV):
            recv = pltpu.make_async_remote_copy(
                src_ref=my_bf,
                dst_ref=wbuf.at[k - 1],
                send_sem=send_sems.at[k - 1],
                recv_sem=recv_sems.at[k - 1],
                device_id=(me,),
                device_id_type=pl.DeviceIdType.MESH,
            )
            recv.wait_recv()
            origin = lax.rem(me + (N_DEV - k), N_DEV)
            out_ref[:, :] += contrib(origin, wbuf.at[k - 1])

        for rdma in rdmas:
            rdma.wait_send()

        @functools.partial(pl.run_scoped, sem=pltpu.SemaphoreType.REGULAR)
        def _(sem):
            for k in range(1, N_DEV):
                pl.semaphore_signal(
                    sem, inc=1,
                    device_id=(lax.rem(me + k, N_DEV),),
                    device_id_type=pl.DeviceIdType.MESH,
                )
            pl.semaphore_wait(sem, N_DEV - 1)

    return pl.pallas_call(
        body,
        out_shape=jax.ShapeDtypeStruct((n_tok, d_out), jnp.float32),
        in_specs=[
            pl.BlockSpec(memory_space=pltpu.VMEM),
            pl.BlockSpec(memory_space=pltpu.VMEM),
            pl.BlockSpec(memory_space=pltpu.VMEM),
            pl.BlockSpec(memory_space=pltpu.VMEM),
        ],
        out_specs=pl.BlockSpec(memory_space=pltpu.VMEM),
        scratch_shapes=[
            pltpu.VMEM((N_DEV - 1, e_per, d_model, d_out), jnp.bfloat16),
            pltpu.VMEM((e_per, d_model, d_out), jnp.bfloat16),
            pltpu.SemaphoreType.DMA((N_DEV - 1,)),
            pltpu.SemaphoreType.DMA((N_DEV - 1,)),
        ],
        compiler_params=pltpu.CompilerParams(
            collective_id=0,
            vmem_limit_bytes=64 * 1024 * 1024,
        ),
    )(x, router_W, route_idx, expert_W)
